# baseline (speedup 1.0000x reference)
import functools

import jax
import jax.numpy as jnp
from jax import lax
from jax.experimental import pallas as pl
from jax.experimental.pallas import tpu as pltpu
from jax.experimental.pallas import tpu_sc as plsc

EMB = 64
PAD = 128
BATCH = 16384
N_SLOTS = 128
SLOT_W = 86400.0 / N_SLOTS

NC = 2
NS = 16
NW = NC * NS
BPW = BATCH // NW
CHUNK = 128
NCH = BPW // CHUNK
LANES = 16
RING = 8
LOOK = 6


def _i16(v):
  return jnp.full((LANES,), v, jnp.int32)


def _body(ss0, ss1, ss2, po0, po1, po2, slots_hbm, t0T, t1T, t2T, ttT,
          out0, out1, out2,
          ssv_v, slv_v, pos2_v, ring_v, out_v, tt_v, gsem):
  c = lax.axis_index("c")
  s = lax.axis_index("s")
  wid = s * NC + c
  base = wid * BPW

  iotas = [lax.iota(jnp.int32, LANES) + 16 * j for j in range(4)]

  def band(ss_hbm, pos_hbm, tT_hbm, out_hbm, with_tt):
    pltpu.sync_copy(ss_hbm.at[pl.ds(base, BPW)], ssv_v.at[pl.ds(0, BPW)])
    for q in range(NCH):
      pltpu.sync_copy(pos_hbm.at[pl.ds(base + q * CHUNK, CHUNK)],
                      pos2_v.at[q])
    if with_tt:
      pltpu.sync_copy(slots_hbm.at[pl.ds(base, BPW)],
                      slv_v.at[pl.ds(0, BPW)])

    def sid(r):
      return ssv_v[pl.ds(r, LANES)][0]

    def blk(r):
      return sid(r) // PAD

    def newblk(r):
      return jnp.where(r == 0, True, blk(r) != blk(jnp.maximum(r - 1, 0)))

    def fire(b, head):
      pltpu.async_copy(tT_hbm.at[:, pl.ds(b * PAD, PAD)],
                       ring_v.at[lax.rem(head, RING)], gsem)

    def drain():
      pltpu.make_async_copy(tT_hbm.at[:, pl.ds(0, PAD)], ring_v.at[0],
                            gsem).wait()

    def pro(r, head):
      @pl.when(newblk(r))
      def _():
        fire(blk(r), head)
      return head + newblk(r).astype(jnp.int32)

    head0 = lax.fori_loop(0, LOOK, pro, jnp.int32(0))

    def step(r, carry):
      head, cur = carry
      pf = jnp.logical_and(r + LOOK < BPW, newblk(r + LOOK))

      @pl.when(pf)
      def _():
        fire(blk(r + LOOK), head)

      head = head + pf.astype(jnp.int32)

      nb = newblk(r)

      @pl.when(nb)
      def _():
        drain()

      cur = cur + nb.astype(jnp.int32)
      slot = lax.rem(cur - 1, RING)

      col = lax.rem(sid(r), PAD)
      r2 = lax.rem(r, CHUNK)
      if with_tt:
        tcol = slv_v[pl.ds(r, LANES)][0]
      for j in range(4):
        val = plsc.load_gather(ring_v, [_i16(slot), iotas[j], _i16(col)])
        if with_tt:
          val = val + plsc.load_gather(tt_v, [iotas[j], _i16(tcol)])
        plsc.store_scatter(out_v, [_i16(r2), iotas[j]], val)

      @pl.when(r2 == CHUNK - 1)
      def _():
        q = r // CHUNK
        pltpu.sync_copy(out_v, out_hbm.at[pos2_v.at[q]])

      return head, cur

    lax.fori_loop(0, BPW, step, (head0, jnp.int32(0)))

  pltpu.sync_copy(ttT, tt_v)
  band(ss0, po0, t0T, out0, False)
  band(ss1, po1, t1T, out1, False)
  band(ss2, po2, t2T, out2, True)


@jax.jit
def kernel(cascades, src_idx, dst_idx, cas_pub_times, user_src_state,
           user_dst_state, cas_state, time_table):
  slot = jnp.clip((cas_pub_times / SLOT_W).astype(jnp.int32), 0, N_SLOTS - 1)
  cas32 = cascades.astype(jnp.int32)
  so = jnp.argsort(src_idx).astype(jnp.int32)
  do = jnp.argsort(dst_idx).astype(jnp.int32)
  co = jnp.argsort(cas32).astype(jnp.int32)
  ss = jnp.take(src_idx, so)
  ds_ = jnp.take(dst_idx, do)
  cs = jnp.take(cas32, co)
  slot_s = jnp.take(slot, co)

  mesh = plsc.VectorSubcoreMesh(core_axis_name="c", subcore_axis_name="s")
  run = pl.kernel(
      _body,
      out_type=[jax.ShapeDtypeStruct((BATCH, PAD), jnp.float32)] * 3,
      mesh=mesh,
      compiler_params=pltpu.CompilerParams(needs_layout_passes=False),
      scratch_types=[
          pltpu.VMEM((BPW + LANES,), jnp.int32),
          pltpu.VMEM((BPW + LANES,), jnp.int32),
          pltpu.VMEM((NCH, CHUNK), jnp.int32),
          pltpu.VMEM((RING, EMB, PAD), jnp.float32),
          pltpu.VMEM((CHUNK, PAD), jnp.float32),
          pltpu.VMEM((EMB, PAD), jnp.float32),
          pltpu.SemaphoreType.DMA,
      ],
  )
  sband, dband, cband = run(ss, ds_, cs, so, do, co, slot_s,
                            user_src_state.T, user_dst_state.T, cas_state.T,
                            time_table.T)
  return jnp.concatenate(
      [sband[:, :EMB], dband[:, :EMB], cband[:, :EMB]], axis=1)


# speedup vs baseline: 1.0687x; 1.0687x over previous
import functools

import jax
import jax.numpy as jnp
from jax import lax
from jax.experimental import pallas as pl
from jax.experimental.pallas import tpu as pltpu
from jax.experimental.pallas import tpu_sc as plsc

EMB = 64
PAD = 128
BATCH = 16384
N_SLOTS = 128
SLOT_W = 86400.0 / N_SLOTS

NC = 2
NS = 16
NW = NC * NS
BPW = BATCH // NW
CHUNK = 128
NCH = BPW // CHUNK
LANES = 16
RING = 8
LOOK = 6


def _i16(v):
  return jnp.full((LANES,), v, jnp.int32)


def _body(ss0, ss1, ss2, po0, po1, po2, slots_hbm, t0T, t1T, t2T, ttT,
          out0, out1, out2,
          ssv_v, slv_v, pos2_v, ring_v, out_v, tt_v, gsem):
  c = lax.axis_index("c")
  s = lax.axis_index("s")
  wid = s * NC + c
  base = wid * BPW

  iotas = [lax.iota(jnp.int32, LANES) + 16 * j for j in range(4)]

  def band(ss_hbm, pos_hbm, tT_hbm, out_hbm, with_tt):
    pltpu.sync_copy(ss_hbm.at[pl.ds(base, BPW)], ssv_v.at[pl.ds(0, BPW)])
    for q in range(NCH):
      pltpu.sync_copy(pos_hbm.at[pl.ds(base + q * CHUNK, CHUNK)],
                      pos2_v.at[q])
    if with_tt:
      pltpu.sync_copy(slots_hbm.at[pl.ds(base, BPW)],
                      slv_v.at[pl.ds(0, BPW)])

    def sid(r):
      return ssv_v[pl.ds(r, LANES)][0]

    def blk(r):
      return sid(r) // PAD

    def newblk(r):
      return jnp.where(r == 0, True, blk(r) != blk(jnp.maximum(r - 1, 0)))

    def fire(b, head):
      pltpu.async_copy(tT_hbm.at[:, pl.ds(b * PAD, PAD)],
                       ring_v.at[lax.rem(head, RING)], gsem)

    def drain():
      pltpu.make_async_copy(tT_hbm.at[:, pl.ds(0, PAD)], ring_v.at[0],
                            gsem).wait()

    def pro(r, head):
      @pl.when(newblk(r))
      def _():
        fire(blk(r), head)
      return head + newblk(r).astype(jnp.int32)

    head0 = lax.fori_loop(0, LOOK, pro, jnp.int32(0))

    def step(r, carry):
      head, cur = carry
      pf = jnp.logical_and(r + LOOK < BPW, newblk(r + LOOK))

      @pl.when(pf)
      def _():
        fire(blk(r + LOOK), head)

      head = head + pf.astype(jnp.int32)

      nb = newblk(r)

      @pl.when(nb)
      def _():
        drain()

      cur = cur + nb.astype(jnp.int32)
      slot = lax.rem(cur - 1, RING)

      col = lax.rem(sid(r), PAD)
      r2 = lax.rem(r, CHUNK)
      if with_tt:
        tcol = slv_v[pl.ds(r, LANES)][0]
      for j in range(4):
        val = plsc.load_gather(ring_v, [_i16(slot), iotas[j], _i16(col)])
        if with_tt:
          val = val + plsc.load_gather(tt_v, [iotas[j], _i16(tcol)])
        plsc.store_scatter(out_v, [_i16(r2), iotas[j]], val)

      @pl.when(r2 == CHUNK - 1)
      def _():
        q = r // CHUNK
        pltpu.sync_copy(out_v, out_hbm.at[pos2_v.at[q]])

      return head, cur

    lax.fori_loop(0, BPW, step, (head0, jnp.int32(0)))

  pltpu.sync_copy(ttT, tt_v)
  band(ss0, po0, t0T, out0, False)
  band(ss1, po1, t1T, out1, False)
  band(ss2, po2, t2T, out2, True)


@jax.jit
def kernel(cascades, src_idx, dst_idx, cas_pub_times, user_src_state,
           user_dst_state, cas_state, time_table):
  slot = jnp.clip((cas_pub_times / SLOT_W).astype(jnp.int32), 0, N_SLOTS - 1)
  cas32 = cascades.astype(jnp.int32)
  pos = lax.iota(jnp.int32, BATCH)
  ss, so = lax.sort((src_idx, pos), num_keys=1, is_stable=False)
  ds_, do = lax.sort((dst_idx, pos), num_keys=1, is_stable=False)
  cs, co, slot_s = lax.sort((cas32, pos, slot), num_keys=1, is_stable=False)

  mesh = plsc.VectorSubcoreMesh(core_axis_name="c", subcore_axis_name="s")
  run = pl.kernel(
      _body,
      out_type=[jax.ShapeDtypeStruct((BATCH, PAD), jnp.float32)] * 3,
      mesh=mesh,
      compiler_params=pltpu.CompilerParams(needs_layout_passes=False),
      scratch_types=[
          pltpu.VMEM((BPW + LANES,), jnp.int32),
          pltpu.VMEM((BPW + LANES,), jnp.int32),
          pltpu.VMEM((NCH, CHUNK), jnp.int32),
          pltpu.VMEM((RING, EMB, PAD), jnp.float32),
          pltpu.VMEM((CHUNK, PAD), jnp.float32),
          pltpu.VMEM((EMB, PAD), jnp.float32),
          pltpu.SemaphoreType.DMA,
      ],
  )
  sband, dband, cband = run(ss, ds_, cs, so, do, co, slot_s,
                            user_src_state.T, user_dst_state.T, cas_state.T,
                            time_table.T)
  return jnp.concatenate(
      [sband[:, :EMB], dband[:, :EMB], cband[:, :EMB]], axis=1)
